# Initial kernel scaffold; baseline (speedup 1.0000x reference)
import jax
import jax.numpy as jnp
from jax.experimental import pallas as pl


def kernel(input, offsets, per_sample_weights, weight):
    raise NotImplementedError("write your pallas kernel here")


# speedup vs baseline: 1.5760x; 1.5760x over previous
import functools

import jax
import jax.numpy as jnp
from jax import lax
from jax.experimental import pallas as pl
from jax.experimental.pallas import tpu as pltpu, tpu_sc as plsc

B = 4096
N = 200
M = 26
DIM = 64
NC = 2
NS = 16
NW = NC * NS
RPW = B // NW
HALF = N // 2
HPAD = 104
LANES = 16
NCH = DIM // LANES


def _body(inp_hbm, offs_hbm, psw_hbm, table_hbm, out_hbm,
          inp_v, offs_v, psw_v, rows_v, cs_v, out_v, sem):
    wid = lax.axis_index("s") * NC + lax.axis_index("c")
    base = wid * RPW

    pltpu.sync_copy(inp_hbm.at[pl.ds(base, RPW)], inp_v)
    pltpu.sync_copy(offs_hbm.at[pl.ds(base, RPW)], offs_v)
    pltpu.sync_copy(psw_hbm.at[pl.ds(base, RPW)], psw_v)

    zero = jnp.zeros((LANES,), jnp.float32)
    lanes = lax.iota(jnp.int32, LANES)

    def splat(x):
        return jnp.full((LANES,), x, jnp.int32)

    def row_body(r, _):
        g0 = pltpu.async_copy(table_hbm.at[inp_v.at[r, 0]], rows_v.at[0], sem)
        g1 = pltpu.async_copy(table_hbm.at[inp_v.at[r, 1]], rows_v.at[1], sem)
        g0.wait()
        g1.wait()

        for c in range(NCH):
            cs_v[0, pl.ds(LANES * c, LANES)] = zero

        def half(k, accs):
            def jb(jj, accs):
                wv = plsc.load_gather(psw_v, [splat(r), splat(k * HALF + jj)])
                out = []
                for c in range(NCH):
                    x = rows_v[k, jj, pl.ds(LANES * c, LANES)]
                    a = accs[c] + x * wv
                    cs_v[k * HALF + jj + 1, pl.ds(LANES * c, LANES)] = a
                    out.append(a)
                return tuple(out)
            return lax.fori_loop(0, HALF, jb, accs)

        accs = half(0, (zero,) * NCH)
        half(1, accs)

        prev = [zero] * NCH
        for m in range(M):
            offm = plsc.load_gather(offs_v, [splat(r), splat(m)]) + 1
            for c in range(NCH):
                cur = plsc.load_gather(cs_v, [offm, lanes + LANES * c])
                out_v[m, pl.ds(LANES * c, LANES)] = cur - prev[c]
                prev[c] = cur
        pltpu.sync_copy(out_v, out_hbm.at[base + r])
        return 0

    lax.fori_loop(0, RPW, row_body, 0)


@functools.partial(
    pl.kernel,
    out_type=jax.ShapeDtypeStruct((B, M, DIM), jnp.float32),
    mesh=plsc.VectorSubcoreMesh(
        core_axis_name="c", subcore_axis_name="s", num_cores=NC, num_subcores=NS
    ),
    scratch_types=[
        pltpu.VMEM((RPW, 2, HPAD), jnp.int32),
        pltpu.VMEM((RPW, M), jnp.int32),
        pltpu.VMEM((RPW, N), jnp.float32),
        pltpu.VMEM((2, HPAD, DIM), jnp.float32),
        pltpu.VMEM((N + 8, DIM), jnp.float32),
        pltpu.VMEM((M, DIM), jnp.float32),
        pltpu.SemaphoreType.DMA,
    ],
    compiler_params=pltpu.CompilerParams(
        use_tc_tiling_on_sc=False, needs_layout_passes=False
    ),
)
def _embedding_bag_sc(inp_hbm, offs_hbm, psw_hbm, table_hbm, out_hbm,
                      inp_v, offs_v, psw_v, rows_v, cs_v, out_v, sem):
    _body(inp_hbm, offs_hbm, psw_hbm, table_hbm, out_hbm,
          inp_v, offs_v, psw_v, rows_v, cs_v, out_v, sem)


def kernel(input, offsets, per_sample_weights, weight):
    inp_pad = jnp.pad(input.reshape(B, 2, HALF), ((0, 0), (0, 0), (0, HPAD - HALF)))
    score = _embedding_bag_sc(inp_pad, offsets, per_sample_weights, weight)
    return score, jnp.float32(0.0)


# speedup vs baseline: 1.7018x; 1.0798x over previous
import functools

import jax
import jax.numpy as jnp
from jax import lax
from jax.experimental import pallas as pl
from jax.experimental.pallas import tpu as pltpu, tpu_sc as plsc

B = 4096
N = 200
M = 26
DIM = 64
NC = 2
NS = 16
NW = NC * NS
RPW = B // NW
HALF = N // 2
HPAD = 104
LANES = 16
NCH = DIM // LANES


def _body(inp_hbm, offs_hbm, psw_hbm, table_hbm, out_hbm,
          inp_v, offs_v, psw_v, rows_v, cs_v, out_v,
          sem0, sem1, osem0, osem1):
    wid = lax.axis_index("s") * NC + lax.axis_index("c")
    base = wid * RPW
    sems = (sem0, sem1)
    osems = (osem0, osem1)

    pltpu.sync_copy(inp_hbm.at[pl.ds(base, RPW)], inp_v)
    pltpu.sync_copy(offs_hbm.at[pl.ds(base, RPW)], offs_v)
    pltpu.sync_copy(psw_hbm.at[pl.ds(base, RPW)], psw_v)

    zero = jnp.zeros((LANES,), jnp.float32)
    lanes = lax.iota(jnp.int32, LANES)

    def splat(x):
        return jnp.full((LANES,), x, jnp.int32)

    for c in range(NCH):
        cs_v[0, pl.ds(LANES * c, LANES)] = zero

    def start_gather(r, buf):
        pltpu.async_copy(table_hbm.at[inp_v.at[r, 0]], rows_v.at[buf, 0], sems[buf])
        pltpu.async_copy(table_hbm.at[inp_v.at[r, 1]], rows_v.at[buf, 1], sems[buf])

    def wait_gather(buf):
        for k in range(2):
            pltpu.make_async_copy(
                table_hbm.at[inp_v.at[0, k]], rows_v.at[buf, k], sems[buf]
            ).wait()

    start_gather(0, 0)

    def g_body(g, _):
        for phase in range(2):
            r = 2 * g + phase
            wait_gather(phase)

            @pl.when(r + 1 < RPW)
            def _():
                start_gather(r + 1, 1 - phase)

            accs = (zero,) * NCH
            for k in range(2):
                def jb(jj, accs, k=k):
                    wv = plsc.load_gather(psw_v, [splat(r), splat(k * HALF + jj)])
                    out = []
                    for c in range(NCH):
                        x = rows_v[phase, k, jj, pl.ds(LANES * c, LANES)]
                        a = accs[c] + x * wv
                        cs_v[k * HALF + jj + 1, pl.ds(LANES * c, LANES)] = a
                        out.append(a)
                    return tuple(out)
                accs = plsc.parallel_loop(0, HALF, unroll=8, carry=accs)(jb)

            @pl.when(g > 0)
            def _():
                pltpu.make_async_copy(
                    out_v.at[phase], out_hbm.at[base], osems[phase]
                ).wait()
            prev = [zero] * NCH
            for m in range(M):
                offm = plsc.load_gather(offs_v, [splat(r), splat(m)]) + 1
                for c in range(NCH):
                    cur = plsc.load_gather(cs_v, [offm, lanes + LANES * c])
                    out_v[phase, m, pl.ds(LANES * c, LANES)] = cur - prev[c]
                    prev[c] = cur
            pltpu.async_copy(out_v.at[phase], out_hbm.at[base + r], osems[phase])
        return 0

    lax.fori_loop(0, RPW // 2, g_body, 0)
    for phase in range(2):
        pltpu.make_async_copy(out_v.at[phase], out_hbm.at[base], osems[phase]).wait()


@functools.partial(
    pl.kernel,
    out_type=jax.ShapeDtypeStruct((B, M, DIM), jnp.float32),
    mesh=plsc.VectorSubcoreMesh(
        core_axis_name="c", subcore_axis_name="s", num_cores=NC, num_subcores=NS
    ),
    scratch_types=[
        pltpu.VMEM((RPW, 2, HPAD), jnp.int32),
        pltpu.VMEM((RPW, M), jnp.int32),
        pltpu.VMEM((RPW, N), jnp.float32),
        pltpu.VMEM((2, 2, HPAD, DIM), jnp.float32),
        pltpu.VMEM((N + 4, DIM), jnp.float32),
        pltpu.VMEM((2, M, DIM), jnp.float32),
        pltpu.SemaphoreType.DMA,
        pltpu.SemaphoreType.DMA,
        pltpu.SemaphoreType.DMA,
        pltpu.SemaphoreType.DMA,
    ],
    compiler_params=pltpu.CompilerParams(
        use_tc_tiling_on_sc=False, needs_layout_passes=False
    ),
)
def _embedding_bag_sc(inp_hbm, offs_hbm, psw_hbm, table_hbm, out_hbm,
                      inp_v, offs_v, psw_v, rows_v, cs_v, out_v,
                      sem0, sem1, osem0, osem1):
    _body(inp_hbm, offs_hbm, psw_hbm, table_hbm, out_hbm,
          inp_v, offs_v, psw_v, rows_v, cs_v, out_v,
          sem0, sem1, osem0, osem1)


def kernel(input, offsets, per_sample_weights, weight):
    inp_pad = jnp.pad(input.reshape(B, 2, HALF), ((0, 0), (0, 0), (0, HPAD - HALF)))
    score = _embedding_bag_sc(inp_pad, offsets, per_sample_weights, weight)
    return score, jnp.float32(0.0)
